# baseline (speedup 1.0000x reference)
import functools

import jax
import jax.numpy as jnp
from jax import lax
from jax.experimental import pallas as pl
from jax.experimental.pallas import tpu as pltpu
from jax.experimental.pallas import tpu_sc as plsc

NUM_ITEMS = 1000000
LATENT_DIM = 64
BATCH = 16384

NC = 2
NS = 16
NW = NC * NS
BPW = BATCH // NW
CHUNK = 128
NCHUNK = BPW // CHUNK

BLKW = 65536
NBLK = (NUM_ITEMS + BLKW - 1) // BLKW


def _matvec_body(w_ref, et_ref, b_ref, y_ref):
    y_ref[...] = jnp.dot(w_ref[...], et_ref[...],
                         preferred_element_type=jnp.float32) + b_ref[0, 0]


def _gather_body(y_hbm, idx_hbm, out_hbm, idx_v, val_v, sem):
    wid = lax.axis_index("s") * NC + lax.axis_index("c")
    base = wid * BPW
    for c in range(NCHUNK):
        pltpu.sync_copy(idx_hbm.at[pl.ds(base + c * CHUNK, CHUNK)],
                        idx_v.at[c])
    copies = []
    for c in range(NCHUNK):
        copies.append(pltpu.async_copy(
            y_hbm.at[idx_v.at[c]],
            val_v.at[pl.ds(c * CHUNK, CHUNK)], sem))
    for cp in copies:
        cp.wait()
    pltpu.sync_copy(val_v, out_hbm.at[pl.ds(base, BPW)])


@jax.jit
def kernel(item_indices, item_emb, ffn_w, ffn_b):
    idx = item_indices.astype(jnp.int32)
    et = jnp.swapaxes(item_emb, 0, 1)
    b2 = ffn_b.reshape(1, 1)

    y2 = pl.pallas_call(
        _matvec_body,
        grid=(NBLK,),
        in_specs=[
            pl.BlockSpec((1, LATENT_DIM), lambda i: (0, 0)),
            pl.BlockSpec((LATENT_DIM, BLKW), lambda i: (0, i)),
            pl.BlockSpec((1, 1), lambda i: (0, 0), memory_space=pltpu.SMEM),
        ],
        out_specs=pl.BlockSpec((1, BLKW), lambda i: (0, i)),
        out_shape=jax.ShapeDtypeStruct((1, NUM_ITEMS), jnp.float32),
    )(ffn_w, et, b2)
    y = y2.reshape(NUM_ITEMS)

    run = pl.kernel(
        _gather_body,
        out_type=jax.ShapeDtypeStruct((BATCH,), jnp.float32),
        mesh=plsc.VectorSubcoreMesh(core_axis_name="c", subcore_axis_name="s",
                                    num_cores=NC, num_subcores=NS),
        compiler_params=pltpu.CompilerParams(needs_layout_passes=False),
        scratch_types=[
            pltpu.VMEM((NCHUNK, CHUNK), jnp.int32),
            pltpu.VMEM((BPW,), jnp.float32),
            pltpu.SemaphoreType.DMA,
        ],
    )
    out = run(y, idx)
    return out.reshape(BATCH, 1)
